# baseline (speedup 1.0000x reference)
import jax
import jax.numpy as jnp
from jax.experimental import pallas as pl
import jax.experimental.pallas.tpu as pltpu

N = 10000
K = 32
F = 128
KF = K * F
BLOCK_N = 1000


def _fold_kernel(w_ref, asrc_ref, atgt_ref, vtgt_ref, bd_ref, wstack_ref):
    W = w_ref[...]
    vtgt_ref[...] = atgt_ref[...] @ W
    vsrc_col = jax.lax.dot_general(W, asrc_ref[...], (((0,), (1,)), ((), ())),
                                   preferred_element_type=jnp.float32)
    k2 = jax.lax.broadcasted_iota(jnp.int32, (K, F, K), 0)
    kk = jax.lax.broadcasted_iota(jnp.int32, (K, F, K), 2)
    vsrc3 = jnp.broadcast_to(vsrc_col, (K, F, K))
    bd_ref[...] = jnp.where(k2 == kk, vsrc3, 0.0).astype(jnp.bfloat16)
    wstack_ref[...] = jnp.broadcast_to(W.T[None], (K, F, F)).astype(jnp.bfloat16)


def _gat_kernel(nodes_ref, nb_ref, vtgt_ref, bd_ref, ws_ref,
                b_ref, out_ref, ex_scr):
    nb2d = nb_ref[...]
    x = nodes_ref[...]

    s = jax.lax.dot_general(nb2d, bd_ref[...], (((1,), (0,)), ((), ())),
                            preferred_element_type=jnp.float32)
    st = (x * vtgt_ref[...]).sum(axis=-1, keepdims=True)

    e = s + st
    e = jnp.where(e > 0, e, 0.2 * e)
    ex = jnp.exp(e)
    denom = ex.sum(axis=-1, keepdims=True) + 1e-16
    ex_scr[...] = ex.astype(jnp.bfloat16)

    zero = pl.program_id(0) * 0
    exb = ex_scr[pl.ds(zero, BLOCK_N), :]

    exr = jnp.repeat(exb, F, axis=1)
    m2 = nb2d * exr

    out = jax.lax.dot_general(m2, ws_ref[...], (((1,), (0,)), ((), ())),
                              preferred_element_type=jnp.float32)
    out = out / denom + b_ref[...]
    out_ref[...] = jnp.where(out > 0, out, jnp.exp(out) - 1.0)


def kernel(nodes, neighbors, W, a_src, a_tgt, b):
    a_src2 = a_src.reshape(1, F)
    a_tgt2 = a_tgt.reshape(1, F)
    b2 = b.reshape(1, F)
    nb2d = neighbors.reshape(N, KF).astype(jnp.bfloat16)

    v_tgt, BD3, Wstack3 = pl.pallas_call(
        _fold_kernel,
        out_shape=[
            jax.ShapeDtypeStruct((1, F), jnp.float32),
            jax.ShapeDtypeStruct((K, F, K), jnp.bfloat16),
            jax.ShapeDtypeStruct((K, F, F), jnp.bfloat16),
        ],
    )(W, a_src2, a_tgt2)
    BD = BD3.reshape(KF, K)
    Wstack = Wstack3.reshape(KF, F)

    grid = (N // BLOCK_N,)
    return pl.pallas_call(
        _gat_kernel,
        grid=grid,
        in_specs=[
            pl.BlockSpec((BLOCK_N, F), lambda i: (i, 0)),
            pl.BlockSpec((BLOCK_N, KF), lambda i: (i, 0)),
            pl.BlockSpec((1, F), lambda i: (0, 0)),
            pl.BlockSpec((KF, K), lambda i: (0, 0)),
            pl.BlockSpec((KF, F), lambda i: (0, 0)),
            pl.BlockSpec((1, F), lambda i: (0, 0)),
        ],
        out_specs=pl.BlockSpec((BLOCK_N, F), lambda i: (i, 0)),
        out_shape=jax.ShapeDtypeStruct((N, F), jnp.float32),
        scratch_shapes=[pltpu.VMEM((BLOCK_N, K), jnp.bfloat16)],
    )(nodes, nb2d, v_tgt, BD, Wstack, b2)


# speedup vs baseline: 1.1038x; 1.1038x over previous
import jax
import jax.numpy as jnp
from jax.experimental import pallas as pl
import jax.experimental.pallas.tpu as pltpu

N = 10000
K = 32
F = 128
KF = K * F
BLOCK_N = 1000


def _fold_kernel(w_ref, asrc_ref, atgt_ref, vtgt_ref, bd_ref):
    W = w_ref[...]
    vtgt_ref[...] = atgt_ref[...] @ W
    vsrc_col = jax.lax.dot_general(W, asrc_ref[...], (((0,), (1,)), ((), ())),
                                   preferred_element_type=jnp.float32)
    k2 = jax.lax.broadcasted_iota(jnp.int32, (K, F, K), 0)
    kk = jax.lax.broadcasted_iota(jnp.int32, (K, F, K), 2)
    vsrc3 = jnp.broadcast_to(vsrc_col, (K, F, K))
    bd_ref[...] = jnp.where(k2 == kk, vsrc3, 0.0)


def _gat_kernel(nodes_ref, nb_ref, w_ref, vtgt_ref, bd_ref, b_ref, out_ref,
                ex_scr):
    nb2d = nb_ref[...]
    x = nodes_ref[...]

    s = jax.lax.dot_general(nb2d, bd_ref[...], (((1,), (0,)), ((), ())),
                            preferred_element_type=jnp.float32)
    st = (x * vtgt_ref[...]).sum(axis=-1, keepdims=True)

    e = s + st
    e = jnp.where(e > 0, e, 0.2 * e)
    ex = jnp.exp(e)
    denom = ex.sum(axis=-1, keepdims=True) + 1e-16
    ex_scr[...] = ex

    zero = (pl.program_id(0) // 1024) * 8
    exd = ex_scr[pl.ds(zero, BLOCK_N), :]

    agg = exd[:, 0:1] * nb2d[:, 0:F]
    for k in range(1, K):
        agg = agg + exd[:, k:k + 1] * nb2d[:, k * F:(k + 1) * F]

    out = jax.lax.dot_general(agg, w_ref[...], (((1,), (1,)), ((), ())),
                              preferred_element_type=jnp.float32)
    out = out / denom + b_ref[...]
    out_ref[...] = jnp.where(out > 0, out, jnp.exp(out) - 1.0)


def kernel(nodes, neighbors, W, a_src, a_tgt, b):
    a_src2 = a_src.reshape(1, F)
    a_tgt2 = a_tgt.reshape(1, F)
    b2 = b.reshape(1, F)
    nb2d = neighbors.reshape(N, KF)

    v_tgt, BD3 = pl.pallas_call(
        _fold_kernel,
        out_shape=[
            jax.ShapeDtypeStruct((1, F), jnp.float32),
            jax.ShapeDtypeStruct((K, F, K), jnp.float32),
        ],
    )(W, a_src2, a_tgt2)
    BD = BD3.reshape(KF, K)

    grid = (N // BLOCK_N,)
    return pl.pallas_call(
        _gat_kernel,
        grid=grid,
        in_specs=[
            pl.BlockSpec((BLOCK_N, F), lambda i: (i, 0)),
            pl.BlockSpec((BLOCK_N, KF), lambda i: (i, 0)),
            pl.BlockSpec((F, F), lambda i: (0, 0)),
            pl.BlockSpec((1, F), lambda i: (0, 0)),
            pl.BlockSpec((KF, K), lambda i: (0, 0)),
            pl.BlockSpec((1, F), lambda i: (0, 0)),
        ],
        out_specs=pl.BlockSpec((BLOCK_N, F), lambda i: (i, 0)),
        out_shape=jax.ShapeDtypeStruct((N, F), jnp.float32),
        scratch_shapes=[pltpu.VMEM((BLOCK_N, K), jnp.float32)],
    )(nodes, nb2d, W, v_tgt, BD, b2)


# speedup vs baseline: 2.4972x; 2.2625x over previous
import jax
import jax.numpy as jnp
from jax.experimental import pallas as pl
import jax.experimental.pallas.tpu as pltpu

N = 10000
K = 32
F = 128
BLOCK_N = 1000


def _fold_vecs_kernel(w_ref, asrc_ref, atgt_ref, vsrc_ref, vtgt_ref):
    W = w_ref[...]
    vsrc_ref[...] = asrc_ref[...] @ W
    vtgt_ref[...] = atgt_ref[...] @ W


def _gat_kernel(nodes_ref, nb_ref, w_ref, vsrc_ref, vtgt_ref, b_ref, out_ref,
                s_scr, st_scr):
    zero = (pl.program_id(0) // 1024) * 8

    nb = nb_ref[...]
    x = nodes_ref[...]
    s_scr[...] = (nb * vsrc_ref[...][None, :, :]).sum(axis=-1)
    st_scr[...] = (x * vtgt_ref[...]).sum(axis=-1, keepdims=True)

    s = s_scr[pl.ds(zero, BLOCK_N), :]
    st = st_scr[pl.ds(zero, BLOCK_N), :]

    e = s + st
    e = jnp.where(e > 0, e, 0.2 * e)
    ex = jnp.exp(e)
    denom = ex.sum(axis=-1, keepdims=True) + 1e-16
    agg = (nb * ex[:, :, None]).sum(axis=1) / denom
    out = jax.lax.dot_general(agg, w_ref[...], (((1,), (1,)), ((), ())),
                              preferred_element_type=jnp.float32)
    out = out + b_ref[...]
    out_ref[...] = jnp.where(out > 0, out, jnp.exp(out) - 1.0)


def kernel(nodes, neighbors, W, a_src, a_tgt, b):
    a_src2 = a_src.reshape(1, F)
    a_tgt2 = a_tgt.reshape(1, F)
    b2 = b.reshape(1, F)

    v_src, v_tgt = pl.pallas_call(
        _fold_vecs_kernel,
        out_shape=[
            jax.ShapeDtypeStruct((1, F), jnp.float32),
            jax.ShapeDtypeStruct((1, F), jnp.float32),
        ],
    )(W, a_src2, a_tgt2)

    grid = (N // BLOCK_N,)
    return pl.pallas_call(
        _gat_kernel,
        grid=grid,
        in_specs=[
            pl.BlockSpec((BLOCK_N, F), lambda i: (i, 0)),
            pl.BlockSpec((BLOCK_N, K, F), lambda i: (i, 0, 0)),
            pl.BlockSpec((F, F), lambda i: (0, 0)),
            pl.BlockSpec((1, F), lambda i: (0, 0)),
            pl.BlockSpec((1, F), lambda i: (0, 0)),
            pl.BlockSpec((1, F), lambda i: (0, 0)),
        ],
        out_specs=pl.BlockSpec((BLOCK_N, F), lambda i: (i, 0)),
        out_shape=jax.ShapeDtypeStruct((N, F), jnp.float32),
        scratch_shapes=[
            pltpu.VMEM((BLOCK_N, K), jnp.float32),
            pltpu.VMEM((BLOCK_N, 1), jnp.float32),
        ],
    )(nodes, neighbors, W, v_src, v_tgt, b2)
